# baseline (speedup 1.0000x reference)
import functools

import jax
import jax.numpy as jnp
from jax import lax
from jax.experimental import pallas as pl
from jax.experimental.pallas import tpu as pltpu
from jax.experimental.pallas import tpu_sc as plsc

_N = 100000
_HID = 32
_HEADS = 2
_E = 1600000

_NC = 2
_NS = 16
_NW = _NC * _NS
_ROWS = 784
_NPAD = _ROWS * 128


def _zero_rows128(ref, n):
    z = jnp.zeros((16,), jnp.float32)

    def body(i, carry):
        for j in range(8):
            ref[i, pl.ds(j * 16, 16)] = z
        return carry

    lax.fori_loop(0, n, body, 0, unroll=2)


def _count_body(dst_hbm, out_hbm, acc, dvb):
    c = lax.axis_index("c")
    s = lax.axis_index("s")
    epb = _E // _NW
    base = (c * _NS + s) * epb
    _zero_rows128(acc, _ROWS)
    ones = jnp.full((16,), 1.0, jnp.float32)

    def chunk(k, carry):
        pltpu.sync_copy(dst_hbm.at[pl.ds(base + k * 2000, 2000)], dvb)

        def inner(j, c2):
            dvv = dvb[pl.ds(j * 16, 16)]
            plsc.addupdate_scatter(acc, [dvv >> 7, dvv & 127], ones)
            return c2

        lax.fori_loop(0, 125, inner, 0, unroll=4)
        return carry

    lax.fori_loop(0, epb // 2000, chunk, 0)
    pltpu.sync_copy(acc, out_hbm.at[c, s])


_NPS = 100096
_CB = 1000


def _spmm16_body(tabs_hbm, sv_hbm, dv_hbm, zero_hbm, out_hbm, svb, dvb, rows, spacc, sem):
    c = lax.axis_index("c")
    s = lax.axis_index("s")
    epb = _E // _NS
    base = s * epb
    tab = tabs_hbm.at[c]

    pltpu.sync_copy(zero_hbm.at[pl.ds(s * 6256, 6256)], spacc.at[pl.ds(s * 6256, 6256)])
    plsc.subcore_barrier()

    def chunk(k, carry):
        pltpu.sync_copy(sv_hbm.at[pl.ds(base + k * _CB, _CB)], svb)
        pltpu.sync_copy(dv_hbm.at[pl.ds(base + k * _CB, _CB)], dvb)
        pltpu.async_copy(tab.at[svb], rows, sem).wait()
        pltpu.sync_copy(rows, spacc.at[dvb], add=True)
        return carry

    lax.fori_loop(0, epb // _CB, chunk, 0)
    plsc.subcore_barrier()
    pltpu.sync_copy(spacc.at[pl.ds(s * 6256, 6256)], out_hbm.at[c, pl.ds(s * 6256, 6256)])


_spmm16_kernel = pl.kernel(
    _spmm16_body,
    out_type=jax.ShapeDtypeStruct((_NC, _NPS, 16), jnp.float32),
    mesh=plsc.VectorSubcoreMesh(core_axis_name="c", subcore_axis_name="s"),
    compiler_params=pltpu.CompilerParams(needs_layout_passes=False, use_tc_tiling_on_sc=False),
    scratch_types=[
        pltpu.VMEM((_CB,), jnp.int32),
        pltpu.VMEM((_CB,), jnp.int32),
        pltpu.VMEM((_CB, 16), jnp.float32),
        pltpu.VMEM_SHARED((_NPS, 16), jnp.float32),
        pltpu.SemaphoreType.DMA,
    ],
)


_CE = 800
_CV = 2000


def _gat_edge_body(as2_hbm, ad2_hbm, sv_hbm, dv_hbm, ex_hbm, den_hbm,
                   svb, dvb, asb, adb, exb, acc):
    c = lax.axis_index("c")
    s = lax.axis_index("s")
    epb = _E // _NS
    base = s * epb
    a_s = as2_hbm.at[c]
    a_d = ad2_hbm.at[c]
    _zero_rows128(acc, _ROWS)

    def chunk(k, carry):
        off = base + k * _CE
        pltpu.sync_copy(sv_hbm.at[pl.ds(off, _CE)], svb)
        pltpu.sync_copy(dv_hbm.at[pl.ds(off, _CE)], dvb)
        pltpu.sync_copy(a_s.at[svb], asb)
        pltpu.sync_copy(a_d.at[dvb], adb)

        def inner(j, c2):
            av = asb[pl.ds(j * 16, 16)] + adb[pl.ds(j * 16, 16)]
            av = jnp.where(av > 0, av, 0.2 * av)
            ex = jnp.exp(av)
            exb[pl.ds(j * 16, 16)] = ex
            dvv = dvb[pl.ds(j * 16, 16)]
            plsc.addupdate_scatter(acc, [dvv >> 7, dvv & 127], ex)
            return c2

        lax.fori_loop(0, _CE // 16, inner, 0, unroll=4)
        pltpu.sync_copy(exb, ex_hbm.at[c, pl.ds(off, _CE)])
        return carry

    lax.fori_loop(0, epb // _CE, chunk, 0)
    pltpu.sync_copy(acc, den_hbm.at[c, s])


_gat_edge_kernel = pl.kernel(
    _gat_edge_body,
    out_type=(jax.ShapeDtypeStruct((_NC, _E), jnp.float32),
              jax.ShapeDtypeStruct((_NC, _NS, _ROWS, 128), jnp.float32)),
    mesh=plsc.VectorSubcoreMesh(core_axis_name="c", subcore_axis_name="s"),
    compiler_params=pltpu.CompilerParams(needs_layout_passes=False, use_tc_tiling_on_sc=False),
    scratch_types=[
        pltpu.VMEM((_CE,), jnp.int32),
        pltpu.VMEM((_CE,), jnp.int32),
        pltpu.VMEM((_CE,), jnp.float32),
        pltpu.VMEM((_CE,), jnp.float32),
        pltpu.VMEM((_CE,), jnp.float32),
        pltpu.VMEM((_ROWS, 128), jnp.float32),
    ],
)


def _gat_msum_body(h2q_hbm, ex_hbm, sv_hbm, dv_hbm, zero_hbm, out_hbm,
                   svb, dvb, exb, rows, spacc, sem):
    c = lax.axis_index("c")
    s = lax.axis_index("s")
    epb = _E // _NS
    base = s * epb
    ex = ex_hbm.at[c]

    for p in range(2):
        q = 2 * c + p
        tab = h2q_hbm.at[q]
        pltpu.sync_copy(zero_hbm.at[pl.ds(s * 6256, 6256)], spacc.at[pl.ds(s * 6256, 6256)])
        plsc.subcore_barrier()

        def chunk(k, carry):
            off = base + k * _CE
            pltpu.sync_copy(sv_hbm.at[pl.ds(off, _CE)], svb)
            pltpu.sync_copy(dv_hbm.at[pl.ds(off, _CE)], dvb)
            pltpu.sync_copy(ex.at[pl.ds(off, _CE)], exb)
            pltpu.async_copy(tab.at[svb], rows, sem).wait()

            def scale(j, c2):
                exv = exb[pl.ds(j * 16, 16)]
                for t in range(16):
                    rows[j * 16 + t] = rows[j * 16 + t] * exv[t]
                return c2

            lax.fori_loop(0, _CE // 16, scale, 0, unroll=2)
            pltpu.sync_copy(rows, spacc.at[dvb], add=True)
            return carry

        lax.fori_loop(0, epb // _CE, chunk, 0)
        plsc.subcore_barrier()
        pltpu.sync_copy(spacc.at[pl.ds(s * 6256, 6256)], out_hbm.at[q, pl.ds(s * 6256, 6256)])
        plsc.subcore_barrier()


_gat_msum_kernel = pl.kernel(
    _gat_msum_body,
    out_type=jax.ShapeDtypeStruct((2 * _NC, _NPS, 16), jnp.float32),
    mesh=plsc.VectorSubcoreMesh(core_axis_name="c", subcore_axis_name="s"),
    compiler_params=pltpu.CompilerParams(needs_layout_passes=False, use_tc_tiling_on_sc=False),
    scratch_types=[
        pltpu.VMEM((_CE,), jnp.int32),
        pltpu.VMEM((_CE,), jnp.int32),
        pltpu.VMEM((_CE,), jnp.float32),
        pltpu.VMEM((_CE, 16), jnp.float32),
        pltpu.VMEM_SHARED((_NPS, 16), jnp.float32),
        pltpu.SemaphoreType.DMA,
    ],
)


def _spmv_body(vals_hbm, sv_hbm, dv_hbm, out_hbm, svb, dvb, vb, acc):
    c = lax.axis_index("c")
    s = lax.axis_index("s")
    epb = _E // _NW
    base = (c * _NS + s) * epb
    _zero_rows128(acc, _ROWS)

    def chunk(k, carry):
        off = base + k * _CV
        pltpu.sync_copy(sv_hbm.at[pl.ds(off, _CV)], svb)
        pltpu.sync_copy(dv_hbm.at[pl.ds(off, _CV)], dvb)
        pltpu.sync_copy(vals_hbm.at[svb], vb)

        def inner(j, c2):
            vv = vb[pl.ds(j * 16, 16)]
            dvv = dvb[pl.ds(j * 16, 16)]
            plsc.addupdate_scatter(acc, [dvv >> 7, dvv & 127], vv)
            return c2

        lax.fori_loop(0, _CV // 16, inner, 0, unroll=4)
        return carry

    lax.fori_loop(0, epb // _CV, chunk, 0)
    pltpu.sync_copy(acc, out_hbm.at[c, s])


_spmv_kernel = pl.kernel(
    _spmv_body,
    out_type=jax.ShapeDtypeStruct((_NC, _NS, _ROWS, 128), jnp.float32),
    mesh=plsc.VectorSubcoreMesh(core_axis_name="c", subcore_axis_name="s"),
    compiler_params=pltpu.CompilerParams(needs_layout_passes=False, use_tc_tiling_on_sc=False),
    scratch_types=[
        pltpu.VMEM((_CV,), jnp.int32),
        pltpu.VMEM((_CV,), jnp.int32),
        pltpu.VMEM((_CV,), jnp.float32),
        pltpu.VMEM((_ROWS, 128), jnp.float32),
    ],
)


_count_kernel = pl.kernel(
    _count_body,
    out_type=jax.ShapeDtypeStruct((_NC, _NS, _ROWS, 128), jnp.float32),
    mesh=plsc.VectorSubcoreMesh(core_axis_name="c", subcore_axis_name="s"),
    compiler_params=pltpu.CompilerParams(needs_layout_passes=False),
    scratch_types=[
        pltpu.VMEM((_ROWS, 128), jnp.float32),
        pltpu.VMEM((2000,), jnp.int32),
    ],
)


_BN = 2000


def _rb(shape, axis):
    def im(i):
        return tuple(i if d == axis else 0 for d in range(len(shape)))
    return pl.BlockSpec(shape, im)


def _wb(shape):
    def im(i):
        return (0,) * len(shape)
    return pl.BlockSpec(shape, im)


def _tc1_body(x_ref, w1_ref, dis_ref, hs1_ref):
    h1 = jnp.dot(x_ref[...], w1_ref[...].T, preferred_element_type=jnp.float32)
    hs1_ref[...] = h1 * dis_ref[...]


def _tc2_body(agg_ref, hs1_ref, dis_ref, b1_ref, wg_ref, atts_ref, attd_ref,
              h2_ref, ast_ref, adt_ref):
    out1 = dis_ref[...] * (agg_ref[...] + hs1_ref[...]) + b1_ref[...]
    r1 = jnp.maximum(out1, 0.0)
    h2 = jnp.dot(r1, wg_ref[...].T, preferred_element_type=jnp.float32)
    h2_ref[...] = h2
    h2r = h2.reshape(-1, _HEADS, _HID)
    ast_ref[...] = (h2r * atts_ref[...]).sum(-1)
    adt_ref[...] = (h2r * attd_ref[...]).sum(-1)


def _tc3_body(msum_ref, den_ref, ast_ref, adt_ref, h2_ref, bg_ref, w2_ref,
              dis_ref, hs3_ref):
    a_self = ast_ref[...] + adt_ref[...]
    ex_self = jnp.exp(jnp.where(a_self > 0, a_self, 0.2 * a_self))
    den = den_ref[...] + ex_self
    h2r = h2_ref[...].reshape(-1, _HEADS, _HID)
    msum = msum_ref[...].reshape(-1, _HEADS, _HID) + h2r * ex_self[:, :, None]
    out2 = (msum / (den[:, :, None] + 1e-16)).mean(axis=1) + bg_ref[...]
    r2 = jnp.maximum(out2, 0.0)
    h3 = jnp.dot(r2, w2_ref[...].T, preferred_element_type=jnp.float32)
    hs3_ref[...] = h3 * dis_ref[...]


def _tc4_body(agg3_ref, hs3_ref, dis_ref, b2_ref, out_ref):
    out_ref[...] = dis_ref[...] * (agg3_ref[...] + hs3_ref[...]) + b2_ref[...]


def _deg2dis_body(degp_ref, dis_ref):
    deg = degp_ref[...].sum(0) + 1.0
    dis_ref[...] = lax.rsqrt(deg)


def _densum_body(denp_ref, den_ref):
    den_ref[...] = denp_ref[...].sum(1)


def _agg3sum_body(aggv_ref, agg3_ref):
    agg3_ref[...] = aggv_ref[...].sum(0)


def kernel(x, edge_index, W1, b1, Wg, att_src, att_dst, bg, W2, b2):
    n = x.shape[0]
    src = edge_index[0]
    dst = edge_index[1]
    f32 = jnp.float32

    degp = _count_kernel(dst).reshape(_NW, _ROWS, 128)
    disp = pl.pallas_call(
        _deg2dis_body,
        grid=(7,),
        in_specs=[_rb((_NW, 112, 128), 1)],
        out_specs=_rb((112, 128), 0),
        out_shape=jax.ShapeDtypeStruct((_ROWS, 128), f32),
    )(degp)
    dis = disp.reshape(_NPAD, 1)[:n]

    hs1 = pl.pallas_call(
        _tc1_body,
        grid=(n // _BN,),
        in_specs=[_rb((_BN, 128), 0), _wb((_HID, 128)), _rb((_BN, 1), 0)],
        out_specs=_rb((_BN, _HID), 0),
        out_shape=jax.ShapeDtypeStruct((n, _HID), f32),
    )(x, W1, dis)
    hs1h = jnp.stack([hs1[:, :16], hs1[:, 16:]])
    zpad = jnp.zeros((_NPS, 16), f32)
    aggp = _spmm16_kernel(hs1h, src, dst, zpad)[:, :n]
    agg1 = jnp.concatenate([aggp[0], aggp[1]], axis=1)

    h2, ast, adt = pl.pallas_call(
        _tc2_body,
        grid=(n // _BN,),
        in_specs=[_rb((_BN, _HID), 0), _rb((_BN, _HID), 0), _rb((_BN, 1), 0),
                  _wb((_HID,)), _wb((2 * _HID, _HID)),
                  _wb((1, _HEADS, _HID)), _wb((1, _HEADS, _HID))],
        out_specs=[_rb((_BN, 2 * _HID), 0), _rb((_BN, 2), 0), _rb((_BN, 2), 0)],
        out_shape=[jax.ShapeDtypeStruct((n, 2 * _HID), f32),
                   jax.ShapeDtypeStruct((n, 2), f32),
                   jax.ShapeDtypeStruct((n, 2), f32)],
    )(agg1, hs1, dis, b1, Wg, att_src, att_dst)
    exh, denp = _gat_edge_kernel(ast.T, adt.T, src, dst)
    denp = denp.reshape(_NC, _NS, _ROWS, 128)
    denh = pl.pallas_call(
        _densum_body,
        grid=(7,),
        in_specs=[_rb((_NC, _NS, 112, 128), 2)],
        out_specs=_rb((_NC, 112, 128), 1),
        out_shape=jax.ShapeDtypeStruct((_NC, _ROWS, 128), f32),
    )(denp)
    den = denh.reshape(_NC, _NPAD)[:, :n].T
    h2q = jnp.stack([h2[:, 16 * q:16 * (q + 1)] for q in range(4)])
    msump = _gat_msum_kernel(h2q, exh, src, dst, zpad)[:, :n]
    msum = jnp.concatenate([msump[q] for q in range(4)], axis=1)
    hs3 = pl.pallas_call(
        _tc3_body,
        grid=(n // _BN,),
        in_specs=[_rb((_BN, 2 * _HID), 0), _rb((_BN, 2), 0), _rb((_BN, 2), 0),
                  _rb((_BN, 2), 0), _rb((_BN, 2 * _HID), 0), _wb((_HID,)),
                  _wb((1, _HID)), _rb((_BN, 1), 0)],
        out_specs=_rb((_BN, 1), 0),
        out_shape=jax.ShapeDtypeStruct((n, 1), f32),
    )(msum, den, ast, adt, h2, bg, W2, dis)

    aggv = _spmv_kernel(hs3.reshape(-1), src, dst).reshape(_NW, _ROWS, 128)
    agg3 = pl.pallas_call(
        _agg3sum_body,
        grid=(7,),
        in_specs=[_rb((_NW, 112, 128), 1)],
        out_specs=_rb((112, 128), 0),
        out_shape=jax.ShapeDtypeStruct((_ROWS, 128), f32),
    )(aggv).reshape(_NPAD, 1)[:n]
    out3 = pl.pallas_call(
        _tc4_body,
        grid=(n // _BN,),
        in_specs=[_rb((_BN, 1), 0), _rb((_BN, 1), 0), _rb((_BN, 1), 0),
                  _wb((1,))],
        out_specs=_rb((_BN, 1), 0),
        out_shape=jax.ShapeDtypeStruct((n, 1), f32),
    )(agg3, hs3, dis, b2)
    return out3.reshape(-1)


# speedup vs baseline: 1.2619x; 1.2619x over previous
import functools

import jax
import jax.numpy as jnp
from jax import lax
from jax.experimental import pallas as pl
from jax.experimental.pallas import tpu as pltpu
from jax.experimental.pallas import tpu_sc as plsc

_N = 100000
_HID = 32
_HEADS = 2
_E = 1600000

_NC = 2
_NS = 16
_NW = _NC * _NS
_ROWS = 784
_NPAD = _ROWS * 128


def _zero_rows128(ref, n):
    z = jnp.zeros((16,), jnp.float32)

    def body(i, carry):
        for j in range(8):
            ref[i, pl.ds(j * 16, 16)] = z
        return carry

    lax.fori_loop(0, n, body, 0, unroll=2)


def _count_body(dst_hbm, out_hbm, acc, dvb):
    c = lax.axis_index("c")
    s = lax.axis_index("s")
    epb = _E // _NW
    base = (c * _NS + s) * epb
    _zero_rows128(acc, _ROWS)
    ones = jnp.full((16,), 1.0, jnp.float32)

    def chunk(k, carry):
        pltpu.sync_copy(dst_hbm.at[pl.ds(base + k * 2000, 2000)], dvb)

        def inner(j, c2):
            dvv = dvb[pl.ds(j * 16, 16)]
            plsc.addupdate_scatter(acc, [dvv >> 7, dvv & 127], ones)
            return c2

        lax.fori_loop(0, 125, inner, 0, unroll=4)
        return carry

    lax.fori_loop(0, epb // 2000, chunk, 0)
    pltpu.sync_copy(acc, out_hbm.at[c, s])


_NPS = 100096
_CB = 1000


def _spmm16_body(tabs_hbm, sv_hbm, dv_hbm, zero_hbm, out_hbm, svb, dvb, rows, spacc, sem):
    c = lax.axis_index("c")
    s = lax.axis_index("s")
    epb = _E // _NS
    base = s * epb
    tab = tabs_hbm.at[c]

    pltpu.sync_copy(zero_hbm.at[pl.ds(s * 6256, 6256)], spacc.at[pl.ds(s * 6256, 6256)])
    plsc.subcore_barrier()

    def chunk(k, carry):
        pltpu.sync_copy(sv_hbm.at[pl.ds(base + k * _CB, _CB)], svb)
        pltpu.sync_copy(dv_hbm.at[pl.ds(base + k * _CB, _CB)], dvb)
        pltpu.async_copy(tab.at[svb], rows, sem).wait()
        pltpu.sync_copy(rows, spacc.at[dvb], add=True)
        return carry

    lax.fori_loop(0, epb // _CB, chunk, 0)
    plsc.subcore_barrier()
    pltpu.sync_copy(spacc.at[pl.ds(s * 6256, 6256)], out_hbm.at[c, pl.ds(s * 6256, 6256)])


_spmm16_kernel = pl.kernel(
    _spmm16_body,
    out_type=jax.ShapeDtypeStruct((_NC, _NPS, 16), jnp.float32),
    mesh=plsc.VectorSubcoreMesh(core_axis_name="c", subcore_axis_name="s"),
    compiler_params=pltpu.CompilerParams(needs_layout_passes=False, use_tc_tiling_on_sc=False),
    scratch_types=[
        pltpu.VMEM((_CB,), jnp.int32),
        pltpu.VMEM((_CB,), jnp.int32),
        pltpu.VMEM((_CB, 16), jnp.float32),
        pltpu.VMEM_SHARED((_NPS, 16), jnp.float32),
        pltpu.SemaphoreType.DMA,
    ],
)


_CE = 800
_CV = 2000


def _gat_edge_body(as2_hbm, ad2_hbm, sv_hbm, dv_hbm, ex_hbm, den_hbm,
                   svb, dvb, asb, adb, exb, acc):
    c = lax.axis_index("c")
    s = lax.axis_index("s")
    epb = _E // _NS
    base = s * epb
    a_s = as2_hbm.at[c]
    a_d = ad2_hbm.at[c]
    _zero_rows128(acc, _ROWS)

    def chunk(k, carry):
        off = base + k * _CE
        pltpu.sync_copy(sv_hbm.at[pl.ds(off, _CE)], svb)
        pltpu.sync_copy(dv_hbm.at[pl.ds(off, _CE)], dvb)
        pltpu.sync_copy(a_s.at[svb], asb)
        pltpu.sync_copy(a_d.at[dvb], adb)

        def inner(j, c2):
            av = asb[pl.ds(j * 16, 16)] + adb[pl.ds(j * 16, 16)]
            av = jnp.where(av > 0, av, 0.2 * av)
            ex = jnp.exp(av)
            exb[pl.ds(j * 16, 16)] = ex
            dvv = dvb[pl.ds(j * 16, 16)]
            plsc.addupdate_scatter(acc, [dvv >> 7, dvv & 127], ex)
            return c2

        lax.fori_loop(0, _CE // 16, inner, 0, unroll=4)
        pltpu.sync_copy(exb, ex_hbm.at[c, pl.ds(off, _CE)])
        return carry

    lax.fori_loop(0, epb // _CE, chunk, 0)
    pltpu.sync_copy(acc, den_hbm.at[c, s])


_gat_edge_kernel = pl.kernel(
    _gat_edge_body,
    out_type=(jax.ShapeDtypeStruct((_NC, _E), jnp.float32),
              jax.ShapeDtypeStruct((_NC, _NS, _ROWS, 128), jnp.float32)),
    mesh=plsc.VectorSubcoreMesh(core_axis_name="c", subcore_axis_name="s"),
    compiler_params=pltpu.CompilerParams(needs_layout_passes=False, use_tc_tiling_on_sc=False),
    scratch_types=[
        pltpu.VMEM((_CE,), jnp.int32),
        pltpu.VMEM((_CE,), jnp.int32),
        pltpu.VMEM((_CE,), jnp.float32),
        pltpu.VMEM((_CE,), jnp.float32),
        pltpu.VMEM((_CE,), jnp.float32),
        pltpu.VMEM((_ROWS, 128), jnp.float32),
    ],
)


def _gat_msum_body(h2q_hbm, ex_hbm, sv_hbm, dv_hbm, zero_hbm, out_hbm,
                   svb, dvb, exb, rows, spacc, sem):
    c = lax.axis_index("c")
    s = lax.axis_index("s")
    epb = _E // _NS
    base = s * epb
    ex = ex_hbm.at[c]

    for p in range(2):
        q = 2 * c + p
        tab = h2q_hbm.at[q]
        pltpu.sync_copy(zero_hbm.at[pl.ds(s * 6256, 6256)], spacc.at[pl.ds(s * 6256, 6256)])
        plsc.subcore_barrier()

        def chunk(k, carry):
            off = base + k * _CE
            pltpu.sync_copy(sv_hbm.at[pl.ds(off, _CE)], svb)
            pltpu.sync_copy(dv_hbm.at[pl.ds(off, _CE)], dvb)
            pltpu.sync_copy(ex.at[pl.ds(off, _CE)], exb)
            pltpu.async_copy(tab.at[svb], rows, sem).wait()

            def scale(j, c2):
                exv = exb[pl.ds(j * 16, 16)]
                for t in range(16):
                    rows[j * 16 + t] = rows[j * 16 + t] * exv[t]
                return c2

            lax.fori_loop(0, _CE // 16, scale, 0, unroll=2)
            pltpu.sync_copy(rows, spacc.at[dvb], add=True)
            return carry

        lax.fori_loop(0, epb // _CE, chunk, 0)
        plsc.subcore_barrier()
        pltpu.sync_copy(spacc.at[pl.ds(s * 6256, 6256)], out_hbm.at[q, pl.ds(s * 6256, 6256)])
        plsc.subcore_barrier()


_gat_msum_kernel = pl.kernel(
    _gat_msum_body,
    out_type=jax.ShapeDtypeStruct((2 * _NC, _NPS, 16), jnp.float32),
    mesh=plsc.VectorSubcoreMesh(core_axis_name="c", subcore_axis_name="s"),
    compiler_params=pltpu.CompilerParams(needs_layout_passes=False, use_tc_tiling_on_sc=False),
    scratch_types=[
        pltpu.VMEM((_CE,), jnp.int32),
        pltpu.VMEM((_CE,), jnp.int32),
        pltpu.VMEM((_CE,), jnp.float32),
        pltpu.VMEM((_CE, 16), jnp.float32),
        pltpu.VMEM_SHARED((_NPS, 16), jnp.float32),
        pltpu.SemaphoreType.DMA,
    ],
)


def _spmv_body(vals_hbm, sv_hbm, dv_hbm, out_hbm, svb, dvb, vb, acc):
    c = lax.axis_index("c")
    s = lax.axis_index("s")
    epb = _E // _NW
    base = (c * _NS + s) * epb
    _zero_rows128(acc, _ROWS)

    def chunk(k, carry):
        off = base + k * _CV
        pltpu.sync_copy(sv_hbm.at[pl.ds(off, _CV)], svb)
        pltpu.sync_copy(dv_hbm.at[pl.ds(off, _CV)], dvb)
        pltpu.sync_copy(vals_hbm.at[svb], vb)

        def inner(j, c2):
            vv = vb[pl.ds(j * 16, 16)]
            dvv = dvb[pl.ds(j * 16, 16)]
            plsc.addupdate_scatter(acc, [dvv >> 7, dvv & 127], vv)
            return c2

        lax.fori_loop(0, _CV // 16, inner, 0, unroll=4)
        return carry

    lax.fori_loop(0, epb // _CV, chunk, 0)
    pltpu.sync_copy(acc, out_hbm.at[c, s])


_spmv_kernel = pl.kernel(
    _spmv_body,
    out_type=jax.ShapeDtypeStruct((_NC, _NS, _ROWS, 128), jnp.float32),
    mesh=plsc.VectorSubcoreMesh(core_axis_name="c", subcore_axis_name="s"),
    compiler_params=pltpu.CompilerParams(needs_layout_passes=False, use_tc_tiling_on_sc=False),
    scratch_types=[
        pltpu.VMEM((_CV,), jnp.int32),
        pltpu.VMEM((_CV,), jnp.int32),
        pltpu.VMEM((_CV,), jnp.float32),
        pltpu.VMEM((_ROWS, 128), jnp.float32),
    ],
)


_count_kernel = pl.kernel(
    _count_body,
    out_type=jax.ShapeDtypeStruct((_NC, _NS, _ROWS, 128), jnp.float32),
    mesh=plsc.VectorSubcoreMesh(core_axis_name="c", subcore_axis_name="s"),
    compiler_params=pltpu.CompilerParams(needs_layout_passes=False),
    scratch_types=[
        pltpu.VMEM((_ROWS, 128), jnp.float32),
        pltpu.VMEM((2000,), jnp.int32),
    ],
)


_BN = 2048


def _rb(shape, axis):
    def im(i):
        return tuple(i if d == axis else 0 for d in range(len(shape)))
    return pl.BlockSpec(shape, im)


def _wb(shape):
    def im(i):
        return (0,) * len(shape)
    return pl.BlockSpec(shape, im)


def _tc1_body(x_ref, w1_ref, dis_ref, hs1h_ref):
    h1 = jnp.dot(x_ref[...], w1_ref[...].T, preferred_element_type=jnp.float32)
    hs1 = h1 * dis_ref[...]
    hs1h_ref[0] = hs1[:, :16]
    hs1h_ref[1] = hs1[:, 16:]


def _tc2_body(agg_ref, hs1h_ref, dis_ref, b1_ref, wg_ref, atts_ref, attd_ref,
              h2q_ref, ast_ref, adt_ref):
    dis = dis_ref[...]
    r10 = jnp.maximum(dis * (agg_ref[0] + hs1h_ref[0]) + b1_ref[0], 0.0)
    r11 = jnp.maximum(dis * (agg_ref[1] + hs1h_ref[1]) + b1_ref[1], 0.0)
    wg = wg_ref[...]
    h2 = (jnp.dot(r10, wg[:, :16].T, preferred_element_type=jnp.float32)
          + jnp.dot(r11, wg[:, 16:].T, preferred_element_type=jnp.float32))
    for q in range(4):
        h2q_ref[q] = h2[:, 16 * q:16 * (q + 1)]
    ast_ref[0] = (h2[:, :32] * atts_ref[0]).sum(axis=1)
    ast_ref[1] = (h2[:, 32:] * atts_ref[1]).sum(axis=1)
    adt_ref[0] = (h2[:, :32] * attd_ref[0]).sum(axis=1)
    adt_ref[1] = (h2[:, 32:] * attd_ref[1]).sum(axis=1)


def _tc3_body(msum_ref, den_ref, ast_ref, adt_ref, h2q_ref, bg_ref, w2_ref,
              dis_ref, hs3_ref):
    a0 = ast_ref[0] + adt_ref[0]
    a1 = ast_ref[1] + adt_ref[1]
    e0 = jnp.exp(jnp.where(a0 > 0, a0, 0.2 * a0))[:, None]
    e1 = jnp.exp(jnp.where(a1 > 0, a1, 0.2 * a1))[:, None]
    d0 = (den_ref[0] + e0[:, 0])[:, None] + 1e-16
    d1 = (den_ref[1] + e1[:, 0])[:, None] + 1e-16
    m0 = (msum_ref[0] + h2q_ref[0] * e0) / d0
    m1 = (msum_ref[1] + h2q_ref[1] * e0) / d0
    m2 = (msum_ref[2] + h2q_ref[2] * e1) / d1
    m3 = (msum_ref[3] + h2q_ref[3] * e1) / d1
    r2a = jnp.maximum(0.5 * (m0 + m2) + bg_ref[0], 0.0)
    r2b = jnp.maximum(0.5 * (m1 + m3) + bg_ref[1], 0.0)
    h3 = (r2a * w2_ref[0]).sum(axis=1) + (r2b * w2_ref[1]).sum(axis=1)
    hs3_ref[...] = h3[:, None] * dis_ref[...]


def _tc4_body(agg3_ref, hs3_ref, dis_ref, b2_ref, out_ref):
    out_ref[...] = dis_ref[...] * (agg3_ref[...] + hs3_ref[...]) + b2_ref[...]


def _deg2dis_body(degp_ref, dis_ref):
    deg = degp_ref[...].sum(0) + 1.0
    dis_ref[...] = lax.rsqrt(deg)


def _densum_body(denp_ref, den_ref):
    den_ref[...] = denp_ref[...].sum(1)


def _agg3sum_body(aggv_ref, agg3_ref):
    agg3_ref[...] = aggv_ref[...].sum(0)


def kernel(x, edge_index, W1, b1, Wg, att_src, att_dst, bg, W2, b2):
    n = x.shape[0]
    src = edge_index[0]
    dst = edge_index[1]
    f32 = jnp.float32
    nb = _NPAD // _BN

    degp = _count_kernel(dst).reshape(_NW, _ROWS, 128)
    disp = pl.pallas_call(
        _deg2dis_body,
        grid=(7,),
        in_specs=[_rb((_NW, 112, 128), 1)],
        out_specs=_rb((112, 128), 0),
        out_shape=jax.ShapeDtypeStruct((_ROWS, 128), f32),
    )(degp).reshape(_NPAD, 1)

    hs1h = pl.pallas_call(
        _tc1_body,
        grid=(nb,),
        in_specs=[_rb((_BN, 128), 0), _wb((_HID, 128)), _rb((_BN, 1), 0)],
        out_specs=_rb((2, _BN, 16), 1),
        out_shape=jax.ShapeDtypeStruct((2, _NPAD, 16), f32),
    )(x, W1, disp)
    zpad = jnp.zeros((_NPS, 16), f32)
    aggp = _spmm16_kernel(hs1h, src, dst, zpad)

    h2q, astT, adtT = pl.pallas_call(
        _tc2_body,
        grid=(nb,),
        in_specs=[_rb((2, _BN, 16), 1), _rb((2, _BN, 16), 1), _rb((_BN, 1), 0),
                  _wb((2, 16)), _wb((2 * _HID, _HID)),
                  _wb((2, _HID)), _wb((2, _HID))],
        out_specs=[_rb((4, _BN, 16), 1), _rb((2, _BN), 1), _rb((2, _BN), 1)],
        out_shape=[jax.ShapeDtypeStruct((4, _NPAD, 16), f32),
                   jax.ShapeDtypeStruct((2, _NPAD), f32),
                   jax.ShapeDtypeStruct((2, _NPAD), f32)],
    )(aggp, hs1h, disp, b1.reshape(2, 16), Wg,
      att_src.reshape(2, _HID), att_dst.reshape(2, _HID))
    exh, denp = _gat_edge_kernel(astT, adtT, src, dst)
    denp = denp.reshape(_NC, _NS, _ROWS, 128)
    denh = pl.pallas_call(
        _densum_body,
        grid=(7,),
        in_specs=[_rb((_NC, _NS, 112, 128), 2)],
        out_specs=_rb((_NC, 112, 128), 1),
        out_shape=jax.ShapeDtypeStruct((_NC, _ROWS, 128), f32),
    )(denp).reshape(_NC, _NPAD)
    msump = _gat_msum_kernel(h2q, exh, src, dst, zpad)
    hs3 = pl.pallas_call(
        _tc3_body,
        grid=(nb,),
        in_specs=[_rb((4, _BN, 16), 1), _rb((2, _BN), 1), _rb((2, _BN), 1),
                  _rb((2, _BN), 1), _rb((4, _BN, 16), 1), _wb((2, 16)),
                  _wb((2, 16)), _rb((_BN, 1), 0)],
        out_specs=_rb((_BN, 1), 0),
        out_shape=jax.ShapeDtypeStruct((_NPAD, 1), f32),
    )(msump, denh, astT, adtT, h2q, bg.reshape(2, 16), W2.reshape(2, 16), disp)

    aggv = _spmv_kernel(hs3.reshape(-1), src, dst).reshape(_NW, _ROWS, 128)
    agg3p = pl.pallas_call(
        _agg3sum_body,
        grid=(7,),
        in_specs=[_rb((_NW, 112, 128), 1)],
        out_specs=_rb((112, 128), 0),
        out_shape=jax.ShapeDtypeStruct((_ROWS, 128), f32),
    )(aggv).reshape(_NPAD, 1)
    out3 = pl.pallas_call(
        _tc4_body,
        grid=(nb,),
        in_specs=[_rb((_BN, 1), 0), _rb((_BN, 1), 0), _rb((_BN, 1), 0),
                  _wb((1,))],
        out_specs=_rb((_BN, 1), 0),
        out_shape=jax.ShapeDtypeStruct((_NPAD, 1), f32),
    )(agg3p, hs3, disp, b2)
    return out3.reshape(-1)[:n]
